# Initial kernel scaffold; baseline (speedup 1.0000x reference)
import jax
import jax.numpy as jnp
from jax.experimental import pallas as pl


def kernel(x, batch, edge_index, snorm_n, edge_attr, emb_h, emb_e, layer_params, readout_params):
    raise NotImplementedError("write your pallas kernel here")


# speedup vs baseline: 1.0206x; 1.0206x over previous
import functools

import jax
import jax.numpy as jnp
from jax import lax
from jax.experimental import pallas as pl
from jax.experimental.pallas import tpu as pltpu
from jax.scipy.linalg import block_diag

N = 50000
E = 800000
HID = 64
TOW = 4
TD = 16
NG = 128
AVG = 2.833
NB = 2000
GRID = N // NB


def _k_embed(x_ref, emb_ref, out_ref):
    x = x_ref[...]
    oh = (x == lax.broadcasted_iota(jnp.int32, (1, 32), 1)).astype(jnp.float32)
    out_ref[...] = jnp.dot(oh, emb_ref[...], preferred_element_type=jnp.float32)


def _embed(x, emb_h):
    embp = jnp.zeros((32, HID), jnp.float32).at[:emb_h.shape[0]].set(emb_h)
    return pl.pallas_call(
        _k_embed,
        grid=(GRID,),
        in_specs=[pl.BlockSpec((NB, 1), lambda i: (i, 0)),
                  pl.BlockSpec((32, HID), lambda i: (0, 0))],
        out_specs=pl.BlockSpec((NB, HID), lambda i: (i, 0)),
        out_shape=jax.ShapeDtypeStruct((N, HID), jnp.float32),
    )(x.reshape(N, 1).astype(jnp.int32), embp)


def _k_node(h_ref, s_ref, s2_ref, mx_ref, mn_ref, deg_ref, sn_ref, wp_ref,
            bp_ref, hm_ref, bn_ref):
    deg = deg_ref[...]
    dd = jnp.where(deg > 0, deg, 1.0)
    slog = jnp.log(deg + 1.0)
    amp = slog / AVG
    att = AVG / jnp.where(slog > 0, slog, 1.0)
    mean = s_ref[...] / dd
    var = jnp.maximum(s2_ref[...] / dd - mean * mean, 0.0)
    std = jnp.sqrt(var + 1e-5)
    msk = deg > 0
    mx = jnp.where(msk, mx_ref[...], 0.0)
    mn = jnp.where(msk, mn_ref[...], 0.0)
    X = jnp.concatenate(
        [h_ref[...], mean, mx, mn, std,
         mean * amp, mx * amp, mn * amp, std * amp,
         mean * att, mx * att, mn * att, std * att], axis=1)
    hm = jnp.dot(X, wp_ref[...], preferred_element_type=jnp.float32) + bp_ref[...]
    hm = jnp.where(hm > 0, hm, 0.01 * hm)
    hm = hm * sn_ref[...]
    hm_ref[...] = hm

    @pl.when(pl.program_id(0) == 0)
    def _():
        bn_ref[...] = jnp.zeros_like(bn_ref)

    bn_ref[...] += jnp.concatenate(
        [jnp.sum(hm, 0, keepdims=True), jnp.sum(hm * hm, 0, keepdims=True)], axis=0)


def _node_stage(h, S, S2, MX, MN, deg2, sn, Wp, bp):
    blk = lambda i: (i, 0)
    fixed = lambda i: (0, 0)
    return pl.pallas_call(
        _k_node,
        grid=(GRID,),
        in_specs=[pl.BlockSpec((NB, HID), blk)] * 5 +
                 [pl.BlockSpec((NB, 1), blk), pl.BlockSpec((NB, 1), blk),
                  pl.BlockSpec((13 * HID, HID), fixed), pl.BlockSpec((1, HID), fixed)],
        out_specs=[pl.BlockSpec((NB, HID), blk), pl.BlockSpec((2, HID), fixed)],
        out_shape=[jax.ShapeDtypeStruct((N, HID), jnp.float32),
                   jax.ShapeDtypeStruct((2, HID), jnp.float32)],
    )(h, S, S2, MX, MN, deg2, sn, Wp, bp)


def _k_bn(hm_ref, hin_ref, bn_ref, g_ref, b_ref, out_ref):
    bn = bn_ref[...]
    mu = bn[0:1, :] / N
    va = bn[1:2, :] / N - mu * mu
    y = (hm_ref[...] - mu) / jnp.sqrt(va + 1e-5) * g_ref[...] + b_ref[...]
    out_ref[...] = hin_ref[...] + jnp.maximum(y, 0.0)


def _bn_stage(hm, h_in, bn, g, b):
    blk = lambda i: (i, 0)
    fixed = lambda i: (0, 0)
    return pl.pallas_call(
        _k_bn,
        grid=(GRID,),
        in_specs=[pl.BlockSpec((NB, HID), blk), pl.BlockSpec((NB, HID), blk),
                  pl.BlockSpec((2, HID), fixed), pl.BlockSpec((1, HID), fixed),
                  pl.BlockSpec((1, HID), fixed)],
        out_specs=pl.BlockSpec((NB, HID), blk),
        out_shape=jax.ShapeDtypeStruct((N, HID), jnp.float32),
    )(hm, h_in, bn, g.reshape(1, HID), b.reshape(1, HID))


def _k_read(h_ref, b_ref, w1, b1, w2, b2, w3, b3, out_ref, hg_acc, cnt_acc):
    i = pl.program_id(0)

    @pl.when(i == 0)
    def _():
        hg_acc[...] = jnp.zeros_like(hg_acc)
        cnt_acc[...] = jnp.zeros_like(cnt_acc)

    bt = b_ref[...]
    oh = (bt == lax.broadcasted_iota(jnp.int32, (1, NG), 1)).astype(jnp.float32)
    dn = (((0,), (0,)), ((), ()))
    hg_acc[...] += lax.dot_general(oh, h_ref[...], dn,
                                   preferred_element_type=jnp.float32)
    cnt_acc[...] += lax.dot_general(oh, jnp.ones((NB, 1), jnp.float32), dn,
                                    preferred_element_type=jnp.float32)

    @pl.when(i == GRID - 1)
    def _():
        cnt = cnt_acc[...]
        hg = hg_acc[...] / jnp.where(cnt > 0, cnt, 1.0)
        z = jnp.maximum(jnp.dot(hg, w1[...], preferred_element_type=jnp.float32)
                        + b1[...], 0.0)
        z = jnp.maximum(jnp.dot(z, w2[...], preferred_element_type=jnp.float32)
                        + b2[...], 0.0)
        out_ref[...] = jnp.dot(z, w3[...], preferred_element_type=jnp.float32) + b3[...]


def _readout(h, batch, rp):
    blk = lambda i: (i, 0)
    fixed = lambda i: (0, 0)
    w1, b1 = rp[0]['w'], rp[0]['b'].reshape(1, -1)
    w2, b2 = rp[1]['w'], rp[1]['b'].reshape(1, -1)
    w3, b3 = rp[2]['w'], rp[2]['b'].reshape(1, -1)
    return pl.pallas_call(
        _k_read,
        grid=(GRID,),
        in_specs=[pl.BlockSpec((NB, HID), blk), pl.BlockSpec((NB, 1), blk),
                  pl.BlockSpec(w1.shape, fixed), pl.BlockSpec(b1.shape, fixed),
                  pl.BlockSpec(w2.shape, fixed), pl.BlockSpec(b2.shape, fixed),
                  pl.BlockSpec(w3.shape, fixed), pl.BlockSpec(b3.shape, fixed)],
        out_specs=pl.BlockSpec((NG, 1), fixed),
        out_shape=jax.ShapeDtypeStruct((NG, 1), jnp.float32),
        scratch_shapes=[pltpu.VMEM((NG, HID), jnp.float32),
                        pltpu.VMEM((NG, 1), jnp.float32)],
    )(h, batch.reshape(N, 1).astype(jnp.int32), w1, b1, w2, b2, w3, b3)


def _fuse_weights(lp):
    mix = lp['mix_w']
    parts = []
    for s in range(13):
        bd = block_diag(*[lp['towers'][t]['post_w'][16 * s:16 * s + 16]
                          for t in range(TOW)])
        parts.append(bd @ mix)
    Wp = jnp.concatenate(parts, axis=0)
    pb = jnp.concatenate([lp['towers'][t]['post_b'] for t in range(TOW)])
    bp = (pb @ mix + lp['mix_b']).reshape(1, HID)
    return Wp, bp


def kernel(x, batch, edge_index, snorm_n, edge_attr, emb_h, emb_e,
           layer_params, readout_params):
    h = _embed(x, emb_h)
    e = jnp.take(emb_e, edge_attr, axis=0)
    src = edge_index[0]
    dst = edge_index[1]
    ones = jnp.ones((E,), jnp.float32)
    deg = jax.ops.segment_sum(ones, dst, num_segments=N)
    deg2 = deg.reshape(N, 1)
    sn = snorm_n.astype(jnp.float32)
    for lp in layer_params:
        Ss, S2s, MXs, MNs = [], [], [], []
        for t in range(TOW):
            tp = lp['towers'][t]
            ht = h[:, 16 * t:16 * t + 16]
            m = jnp.concatenate([ht[dst], ht[src], e], axis=1) @ tp['pre_w'] + tp['pre_b']
            Ss.append(jax.ops.segment_sum(m, dst, num_segments=N))
            S2s.append(jax.ops.segment_sum(m * m, dst, num_segments=N))
            MXs.append(jax.ops.segment_max(m, dst, num_segments=N))
            MNs.append(jax.ops.segment_min(m, dst, num_segments=N))
        S = jnp.concatenate(Ss, axis=1)
        S2 = jnp.concatenate(S2s, axis=1)
        MX = jnp.concatenate(MXs, axis=1)
        MN = jnp.concatenate(MNs, axis=1)
        Wp, bp = _fuse_weights(lp)
        hm, bn = _node_stage(h, S, S2, MX, MN, deg2, sn, Wp, bp)
        h = _bn_stage(hm, h, bn, lp['bn_g'], lp['bn_b'])
    return _readout(h, batch, readout_params)


# speedup vs baseline: 10.9579x; 10.7367x over previous
import functools

import jax
import jax.numpy as jnp
from jax import lax
from jax.experimental import pallas as pl
from jax.experimental.pallas import tpu as pltpu
from jax.experimental.pallas import tpu_sc as plsc
from jax.scipy.linalg import block_diag

N = 50000
E = 800000
HID = 64
TOW = 4
TD = 16
NG = 128
AVG = 2.833
NB = 2000
GRID = N // NB

NSUB = 313
NCUT = 160
NPAD = NSUB * NCUT
CE = 256
FMAX = 3e38


def _k_embed(x_ref, emb_ref, out_ref):
    x = x_ref[...]
    oh = (x == lax.broadcasted_iota(jnp.int32, (1, 32), 1)).astype(jnp.float32)
    out_ref[...] = jnp.dot(oh, emb_ref[...], preferred_element_type=jnp.float32)


def _embed(x, emb_h):
    embp = jnp.zeros((32, HID), jnp.float32).at[:emb_h.shape[0]].set(emb_h)
    return pl.pallas_call(
        _k_embed,
        grid=(GRID,),
        in_specs=[pl.BlockSpec((NB, 1), lambda i: (i, 0)),
                  pl.BlockSpec((32, HID), lambda i: (0, 0))],
        out_specs=pl.BlockSpec((NB, HID), lambda i: (i, 0)),
        out_shape=jax.ShapeDtypeStruct((N, HID), jnp.float32),
    )(x.reshape(N, 1).astype(jnp.int32), embp)


def _k_node(h_ref, s_ref, s2_ref, mx_ref, mn_ref, deg_ref, sn_ref, wp_ref,
            bp_ref, hm_ref, bn_ref):
    deg = deg_ref[...]
    dd = jnp.where(deg > 0, deg, 1.0)
    slog = jnp.log(deg + 1.0)
    amp = slog / AVG
    att = AVG / jnp.where(slog > 0, slog, 1.0)
    mean = s_ref[...] / dd
    var = jnp.maximum(s2_ref[...] / dd - mean * mean, 0.0)
    std = jnp.sqrt(var + 1e-5)
    msk = deg > 0
    mx = jnp.where(msk, mx_ref[...], 0.0)
    mn = jnp.where(msk, mn_ref[...], 0.0)
    X = jnp.concatenate(
        [h_ref[...], mean, mx, mn, std,
         mean * amp, mx * amp, mn * amp, std * amp,
         mean * att, mx * att, mn * att, std * att], axis=1)
    hm = jnp.dot(X, wp_ref[...], preferred_element_type=jnp.float32) + bp_ref[...]
    hm = jnp.where(hm > 0, hm, 0.01 * hm)
    hm = hm * sn_ref[...]
    hm_ref[...] = hm

    @pl.when(pl.program_id(0) == 0)
    def _():
        bn_ref[...] = jnp.zeros_like(bn_ref)

    bn_ref[...] += jnp.concatenate(
        [jnp.sum(hm, 0, keepdims=True), jnp.sum(hm * hm, 0, keepdims=True)], axis=0)


def _node_stage(h, S, S2, MX, MN, deg2, sn, Wp, bp):
    blk = lambda i: (i, 0)
    fixed = lambda i: (0, 0)
    return pl.pallas_call(
        _k_node,
        grid=(GRID,),
        in_specs=[pl.BlockSpec((NB, HID), blk)] * 5 +
                 [pl.BlockSpec((NB, 1), blk), pl.BlockSpec((NB, 1), blk),
                  pl.BlockSpec((13 * HID, HID), fixed), pl.BlockSpec((1, HID), fixed)],
        out_specs=[pl.BlockSpec((NB, HID), blk), pl.BlockSpec((2, HID), fixed)],
        out_shape=[jax.ShapeDtypeStruct((N, HID), jnp.float32),
                   jax.ShapeDtypeStruct((2, HID), jnp.float32)],
    )(h, S, S2, MX, MN, deg2, sn, Wp, bp)


def _k_bn(hm_ref, hin_ref, bn_ref, g_ref, b_ref, out_ref):
    bn = bn_ref[...]
    mu = bn[0:1, :] / N
    va = bn[1:2, :] / N - mu * mu
    y = (hm_ref[...] - mu) / jnp.sqrt(va + 1e-5) * g_ref[...] + b_ref[...]
    out_ref[...] = hin_ref[...] + jnp.maximum(y, 0.0)


def _bn_stage(hm, h_in, bn, g, b):
    blk = lambda i: (i, 0)
    fixed = lambda i: (0, 0)
    return pl.pallas_call(
        _k_bn,
        grid=(GRID,),
        in_specs=[pl.BlockSpec((NB, HID), blk), pl.BlockSpec((NB, HID), blk),
                  pl.BlockSpec((2, HID), fixed), pl.BlockSpec((1, HID), fixed),
                  pl.BlockSpec((1, HID), fixed)],
        out_specs=pl.BlockSpec((NB, HID), blk),
        out_shape=jax.ShapeDtypeStruct((N, HID), jnp.float32),
    )(hm, h_in, bn, g.reshape(1, HID), b.reshape(1, HID))


def _k_read(h_ref, b_ref, w1, b1, w2, b2, w3, b3, out_ref, hg_acc, cnt_acc):
    i = pl.program_id(0)

    @pl.when(i == 0)
    def _():
        hg_acc[...] = jnp.zeros_like(hg_acc)
        cnt_acc[...] = jnp.zeros_like(cnt_acc)

    bt = b_ref[...]
    oh = (bt == lax.broadcasted_iota(jnp.int32, (1, NG), 1)).astype(jnp.float32)
    dn = (((0,), (0,)), ((), ()))
    hg_acc[...] += lax.dot_general(oh, h_ref[...], dn,
                                   preferred_element_type=jnp.float32)
    cnt_acc[...] += lax.dot_general(oh, jnp.ones((NB, 1), jnp.float32), dn,
                                    preferred_element_type=jnp.float32)

    @pl.when(i == GRID - 1)
    def _():
        cnt = cnt_acc[...]
        hg = hg_acc[...] / jnp.where(cnt > 0, cnt, 1.0)
        z = jnp.maximum(jnp.dot(hg, w1[...], preferred_element_type=jnp.float32)
                        + b1[...], 0.0)
        z = jnp.maximum(jnp.dot(z, w2[...], preferred_element_type=jnp.float32)
                        + b2[...], 0.0)
        out_ref[...] = jnp.dot(z, w3[...], preferred_element_type=jnp.float32) + b3[...]


def _readout(h, batch, rp):
    blk = lambda i: (i, 0)
    fixed = lambda i: (0, 0)
    w1, b1 = rp[0]['w'], rp[0]['b'].reshape(1, -1)
    w2, b2 = rp[1]['w'], rp[1]['b'].reshape(1, -1)
    w3, b3 = rp[2]['w'], rp[2]['b'].reshape(1, -1)
    return pl.pallas_call(
        _k_read,
        grid=(GRID,),
        in_specs=[pl.BlockSpec((NB, HID), blk), pl.BlockSpec((NB, 1), blk),
                  pl.BlockSpec(w1.shape, fixed), pl.BlockSpec(b1.shape, fixed),
                  pl.BlockSpec(w2.shape, fixed), pl.BlockSpec(b2.shape, fixed),
                  pl.BlockSpec(w3.shape, fixed), pl.BlockSpec(b3.shape, fixed)],
        out_specs=pl.BlockSpec((NG, 1), fixed),
        out_shape=jax.ShapeDtypeStruct((NG, 1), jnp.float32),
        scratch_shapes=[pltpu.VMEM((NG, HID), jnp.float32),
                        pltpu.VMEM((NG, 1), jnp.float32)],
    )(h, batch.reshape(N, 1).astype(jnp.int32), w1, b1, w2, b2, w3, b3)


def _k_ptab(h_ref, wd_ref, ws_ref, embe_ref, wec_ref, pbc_ref,
            pd_ref, ps_ref, pe_ref):
    pd_ref[...] = jnp.dot(h_ref[...], wd_ref[...], preferred_element_type=jnp.float32)
    ps_ref[...] = jnp.dot(h_ref[...], ws_ref[...], preferred_element_type=jnp.float32)

    @pl.when(pl.program_id(0) == 0)
    def _():
        pe_ref[...] = jnp.dot(embe_ref[...], wec_ref[...],
                              preferred_element_type=jnp.float32) + pbc_ref[...]


def _ptables(h, Wd, Ws, embe_pad, wec, pbc):
    blk = lambda i: (i, 0)
    fixed = lambda i: (0, 0)
    return pl.pallas_call(
        _k_ptab,
        grid=(GRID,),
        in_specs=[pl.BlockSpec((NB, HID), blk), pl.BlockSpec((HID, HID), fixed),
                  pl.BlockSpec((HID, HID), fixed), pl.BlockSpec((8, 16), fixed),
                  pl.BlockSpec((16, HID), fixed), pl.BlockSpec((1, HID), fixed)],
        out_specs=[pl.BlockSpec((NB, HID), blk), pl.BlockSpec((NB, HID), blk),
                   pl.BlockSpec((8, HID), fixed)],
        out_shape=[jax.ShapeDtypeStruct((N, HID), jnp.float32),
                   jax.ShapeDtypeStruct((N, HID), jnp.float32),
                   jax.ShapeDtypeStruct((8, HID), jnp.float32)],
    )(h, Wd, Ws, embe_pad, wec, pbc)


_SC_MESH = plsc.VectorSubcoreMesh(core_axis_name="c", subcore_axis_name="s")


@functools.partial(
    pl.kernel, mesh=_SC_MESH,
    compiler_params=pltpu.CompilerParams(use_tc_tiling_on_sc=False),
    out_type=[jax.ShapeDtypeStruct((NPAD * HID,), jnp.float32)] * 4,
    scratch_types=[
        pltpu.VMEM((NSUB * HID,), jnp.float32),
        pltpu.VMEM((NSUB * HID,), jnp.float32),
        pltpu.VMEM((NSUB * HID,), jnp.float32),
        pltpu.VMEM((NSUB * HID,), jnp.float32),
        pltpu.VMEM((CE + 16,), jnp.int32),
        pltpu.VMEM((CE,), jnp.int32),
        pltpu.VMEM((CE + 16,), jnp.int32),
        pltpu.VMEM((CE, HID), jnp.float32),
        pltpu.VMEM((CE, HID), jnp.float32),
        pltpu.VMEM((184,), jnp.int32),
        pltpu.VMEM((8, HID), jnp.float32),
        pltpu.SemaphoreType.DMA,
        pltpu.SemaphoreType.DMA,
    ])
def _sc_edge(pd_hbm, ps_hbm, pe_hbm, dst_hbm, src_hbm, attr_hbm, est_hbm,
             s_hbm, s2_hbm, mx_hbm, mn_hbm,
             sacc, s2acc, mxacc, mnacc, dstv, srcv, attrv, drows, srows,
             estv, pev, sem1, sem2):
    wid = lax.axis_index("s") * 2 + lax.axis_index("c")
    pltpu.sync_copy(est_hbm, estv)
    pltpu.sync_copy(pe_hbm, pev)
    zero16 = jnp.zeros((16,), jnp.float32)
    neg16 = jnp.full((16,), -FMAX, jnp.float32)
    pos16 = jnp.full((16,), FMAX, jnp.float32)
    for r in range(5):
        j = wid * 5 + r
        ev = estv[pl.ds(j, 16)]
        elo = ev[0]
        ehi = ev[1]

        def initbody(i, _):
            o = pl.ds(i * 16, 16)
            sacc[o] = zero16
            s2acc[o] = zero16
            mxacc[o] = neg16
            mnacc[o] = pos16
            return 0

        lax.fori_loop(0, NSUB * 4, initbody, 0)
        base0 = (elo // CE) * CE
        nch = (ehi - base0 + CE - 1) // CE
        nlo = j * NSUB

        def chunk(ci, _):
            base = base0 + ci * CE
            pltpu.sync_copy(dst_hbm.at[pl.ds(base, CE)], dstv.at[pl.ds(0, CE)])
            pltpu.sync_copy(src_hbm.at[pl.ds(base, CE)], srcv)
            pltpu.sync_copy(attr_hbm.at[pl.ds(base, CE)], attrv.at[pl.ds(0, CE)])
            cpd = pltpu.async_copy(pd_hbm.at[dstv.at[pl.ds(0, CE)]], drows, sem1)
            cps = pltpu.async_copy(ps_hbm.at[srcv], srows, sem2)
            cpd.wait()
            cps.wait()
            lo_i = jnp.maximum(elo - base, 0)
            hi_i = jnp.minimum(ehi - base, CE)

            def edge(c, _):
                a = attrv[pl.ds(c, 16)][0]
                d = dstv[pl.ds(c, 16)][0]
                off = (d - nlo) * HID
                for k in range(4):
                    mk = (drows[c, pl.ds(16 * k, 16)]
                          + srows[c, pl.ds(16 * k, 16)]
                          + pev[a, pl.ds(16 * k, 16)])
                    o = pl.ds(off + 16 * k, 16)
                    sacc[o] += mk
                    s2acc[o] += mk * mk
                    mxacc[o] = jnp.maximum(mxacc[o], mk)
                    mnacc[o] = jnp.minimum(mnacc[o], mk)
                return 0

            lax.fori_loop(lo_i, hi_i, edge, 0)
            return 0

        lax.fori_loop(0, nch, chunk, 0)
        oo = pl.ds(j * NSUB * HID, NSUB * HID)
        pltpu.sync_copy(sacc, s_hbm.at[oo])
        pltpu.sync_copy(s2acc, s2_hbm.at[oo])
        pltpu.sync_copy(mxacc, mx_hbm.at[oo])
        pltpu.sync_copy(mnacc, mn_hbm.at[oo])


def _fuse_weights(lp):
    mix = lp['mix_w']
    parts = []
    for s in range(13):
        bd = block_diag(*[lp['towers'][t]['post_w'][16 * s:16 * s + 16]
                          for t in range(TOW)])
        parts.append(bd @ mix)
    Wp = jnp.concatenate(parts, axis=0)
    pb = jnp.concatenate([lp['towers'][t]['post_b'] for t in range(TOW)])
    bp = (pb @ mix + lp['mix_b']).reshape(1, HID)
    return Wp, bp


def _pre_weights(lp, emb_e):
    Wd = block_diag(*[lp['towers'][t]['pre_w'][0:16] for t in range(TOW)])
    Ws = block_diag(*[lp['towers'][t]['pre_w'][16:32] for t in range(TOW)])
    wec = jnp.concatenate([lp['towers'][t]['pre_w'][32:48] for t in range(TOW)],
                          axis=1)
    pbc = jnp.concatenate([lp['towers'][t]['pre_b'] for t in range(TOW)]).reshape(1, HID)
    embe_pad = jnp.zeros((8, 16), jnp.float32).at[:emb_e.shape[0]].set(emb_e)
    return Wd, Ws, embe_pad, wec, pbc


def kernel(x, batch, edge_index, snorm_n, edge_attr, emb_h, emb_e,
           layer_params, readout_params):
    h = _embed(x, emb_h)
    src = edge_index[0].astype(jnp.int32)
    dst = edge_index[1].astype(jnp.int32)
    perm = jnp.argsort(dst)
    dst_s = jnp.take(dst, perm)
    src_s = jnp.take(src, perm)
    attr_s = jnp.take(edge_attr.astype(jnp.int32), perm)
    row_starts = jnp.searchsorted(dst_s, jnp.arange(N + 1, dtype=jnp.int32)
                                  ).astype(jnp.int32)
    deg2 = (row_starts[1:] - row_starts[:-1]).astype(jnp.float32).reshape(N, 1)
    cuts = jnp.minimum(jnp.arange(NCUT + 1, dtype=jnp.int32) * NSUB, N)
    est = jnp.take(row_starts, cuts)
    est_p = jnp.full((184,), E, jnp.int32).at[:NCUT + 1].set(est)
    padi = jnp.zeros((CE,), jnp.int32)
    dst_p = jnp.concatenate([dst_s, padi])
    src_p = jnp.concatenate([src_s, padi])
    attr_p = jnp.concatenate([attr_s, padi])
    sn = snorm_n.astype(jnp.float32)
    for lp in layer_params:
        Wd, Ws, embe_pad, wec, pbc = _pre_weights(lp, emb_e)
        pd, ps, pe = _ptables(h, Wd, Ws, embe_pad, wec, pbc)
        Sf, S2f, MXf, MNf = _sc_edge(pd, ps, pe, dst_p, src_p, attr_p, est_p)
        S = Sf[:N * HID].reshape(N, HID)
        S2 = S2f[:N * HID].reshape(N, HID)
        MX = MXf[:N * HID].reshape(N, HID)
        MN = MNf[:N * HID].reshape(N, HID)
        Wp, bp = _fuse_weights(lp)
        hm, bn = _node_stage(h, S, S2, MX, MN, deg2, sn, Wp, bp)
        h = _bn_stage(hm, h, bn, lp['bn_g'], lp['bn_b'])
    return _readout(h, batch, readout_params)


# speedup vs baseline: 11.7101x; 1.0686x over previous
import functools

import jax
import jax.numpy as jnp
from jax import lax
from jax.experimental import pallas as pl
from jax.experimental.pallas import tpu as pltpu
from jax.experimental.pallas import tpu_sc as plsc
from jax.scipy.linalg import block_diag

N = 50000
E = 800000
HID = 64
TOW = 4
TD = 16
NG = 128
AVG = 2.833
NB = 2000
GRID = N // NB

NSUB = 264
NCUT = 192
NPAD = NSUB * NCUT
CE = 256
FMAX = 3e38


def _k_embed(x_ref, emb_ref, out_ref):
    x = x_ref[...]
    oh = (x == lax.broadcasted_iota(jnp.int32, (1, 32), 1)).astype(jnp.float32)
    out_ref[...] = jnp.dot(oh, emb_ref[...], preferred_element_type=jnp.float32)


def _embed(x, emb_h):
    embp = jnp.zeros((32, HID), jnp.float32).at[:emb_h.shape[0]].set(emb_h)
    return pl.pallas_call(
        _k_embed,
        grid=(GRID,),
        in_specs=[pl.BlockSpec((NB, 1), lambda i: (i, 0)),
                  pl.BlockSpec((32, HID), lambda i: (0, 0))],
        out_specs=pl.BlockSpec((NB, HID), lambda i: (i, 0)),
        out_shape=jax.ShapeDtypeStruct((N, HID), jnp.float32),
    )(x.reshape(N, 1).astype(jnp.int32), embp)


def _k_node(h_ref, s_ref, s2_ref, mx_ref, mn_ref, deg_ref, sn_ref, wp_ref,
            bp_ref, hm_ref, bn_ref):
    deg = deg_ref[...]
    dd = jnp.where(deg > 0, deg, 1.0)
    slog = jnp.log(deg + 1.0)
    amp = slog / AVG
    att = AVG / jnp.where(slog > 0, slog, 1.0)
    mean = s_ref[...] / dd
    var = jnp.maximum(s2_ref[...] / dd - mean * mean, 0.0)
    std = jnp.sqrt(var + 1e-5)
    msk = deg > 0
    mx = jnp.where(msk, mx_ref[...], 0.0)
    mn = jnp.where(msk, mn_ref[...], 0.0)
    X = jnp.concatenate(
        [h_ref[...], mean, mx, mn, std,
         mean * amp, mx * amp, mn * amp, std * amp,
         mean * att, mx * att, mn * att, std * att], axis=1)
    hm = jnp.dot(X, wp_ref[...], preferred_element_type=jnp.float32) + bp_ref[...]
    hm = jnp.where(hm > 0, hm, 0.01 * hm)
    hm = hm * sn_ref[...]
    hm_ref[...] = hm

    @pl.when(pl.program_id(0) == 0)
    def _():
        bn_ref[...] = jnp.zeros_like(bn_ref)

    bn_ref[...] += jnp.concatenate(
        [jnp.sum(hm, 0, keepdims=True), jnp.sum(hm * hm, 0, keepdims=True)], axis=0)


def _node_stage(h, S, S2, MX, MN, deg2, sn, Wp, bp):
    blk = lambda i: (i, 0)
    fixed = lambda i: (0, 0)
    return pl.pallas_call(
        _k_node,
        grid=(GRID,),
        in_specs=[pl.BlockSpec((NB, HID), blk)] * 5 +
                 [pl.BlockSpec((NB, 1), blk), pl.BlockSpec((NB, 1), blk),
                  pl.BlockSpec((13 * HID, HID), fixed), pl.BlockSpec((1, HID), fixed)],
        out_specs=[pl.BlockSpec((NB, HID), blk), pl.BlockSpec((2, HID), fixed)],
        out_shape=[jax.ShapeDtypeStruct((N, HID), jnp.float32),
                   jax.ShapeDtypeStruct((2, HID), jnp.float32)],
    )(h, S, S2, MX, MN, deg2, sn, Wp, bp)


def _k_bn(hm_ref, hin_ref, bn_ref, g_ref, b_ref, out_ref):
    bn = bn_ref[...]
    mu = bn[0:1, :] / N
    va = bn[1:2, :] / N - mu * mu
    y = (hm_ref[...] - mu) / jnp.sqrt(va + 1e-5) * g_ref[...] + b_ref[...]
    out_ref[...] = hin_ref[...] + jnp.maximum(y, 0.0)


def _bn_stage(hm, h_in, bn, g, b):
    blk = lambda i: (i, 0)
    fixed = lambda i: (0, 0)
    return pl.pallas_call(
        _k_bn,
        grid=(GRID,),
        in_specs=[pl.BlockSpec((NB, HID), blk), pl.BlockSpec((NB, HID), blk),
                  pl.BlockSpec((2, HID), fixed), pl.BlockSpec((1, HID), fixed),
                  pl.BlockSpec((1, HID), fixed)],
        out_specs=pl.BlockSpec((NB, HID), blk),
        out_shape=jax.ShapeDtypeStruct((N, HID), jnp.float32),
    )(hm, h_in, bn, g.reshape(1, HID), b.reshape(1, HID))


def _k_read(h_ref, b_ref, w1, b1, w2, b2, w3, b3, out_ref, hg_acc, cnt_acc):
    i = pl.program_id(0)

    @pl.when(i == 0)
    def _():
        hg_acc[...] = jnp.zeros_like(hg_acc)
        cnt_acc[...] = jnp.zeros_like(cnt_acc)

    bt = b_ref[...]
    oh = (bt == lax.broadcasted_iota(jnp.int32, (1, NG), 1)).astype(jnp.float32)
    dn = (((0,), (0,)), ((), ()))
    hg_acc[...] += lax.dot_general(oh, h_ref[...], dn,
                                   preferred_element_type=jnp.float32)
    cnt_acc[...] += lax.dot_general(oh, jnp.ones((NB, 1), jnp.float32), dn,
                                    preferred_element_type=jnp.float32)

    @pl.when(i == GRID - 1)
    def _():
        cnt = cnt_acc[...]
        hg = hg_acc[...] / jnp.where(cnt > 0, cnt, 1.0)
        z = jnp.maximum(jnp.dot(hg, w1[...], preferred_element_type=jnp.float32)
                        + b1[...], 0.0)
        z = jnp.maximum(jnp.dot(z, w2[...], preferred_element_type=jnp.float32)
                        + b2[...], 0.0)
        out_ref[...] = jnp.dot(z, w3[...], preferred_element_type=jnp.float32) + b3[...]


def _readout(h, batch, rp):
    blk = lambda i: (i, 0)
    fixed = lambda i: (0, 0)
    w1, b1 = rp[0]['w'], rp[0]['b'].reshape(1, -1)
    w2, b2 = rp[1]['w'], rp[1]['b'].reshape(1, -1)
    w3, b3 = rp[2]['w'], rp[2]['b'].reshape(1, -1)
    return pl.pallas_call(
        _k_read,
        grid=(GRID,),
        in_specs=[pl.BlockSpec((NB, HID), blk), pl.BlockSpec((NB, 1), blk),
                  pl.BlockSpec(w1.shape, fixed), pl.BlockSpec(b1.shape, fixed),
                  pl.BlockSpec(w2.shape, fixed), pl.BlockSpec(b2.shape, fixed),
                  pl.BlockSpec(w3.shape, fixed), pl.BlockSpec(b3.shape, fixed)],
        out_specs=pl.BlockSpec((NG, 1), fixed),
        out_shape=jax.ShapeDtypeStruct((NG, 1), jnp.float32),
        scratch_shapes=[pltpu.VMEM((NG, HID), jnp.float32),
                        pltpu.VMEM((NG, 1), jnp.float32)],
    )(h, batch.reshape(N, 1).astype(jnp.int32), w1, b1, w2, b2, w3, b3)


def _k_ptab(h_ref, wd_ref, ws_ref, embe_ref, wec_ref, pbc_ref,
            pd_ref, ps_ref, pe_ref):
    pd_ref[...] = jnp.dot(h_ref[...], wd_ref[...], preferred_element_type=jnp.float32)
    ps_ref[...] = jnp.dot(h_ref[...], ws_ref[...], preferred_element_type=jnp.float32)

    @pl.when(pl.program_id(0) == 0)
    def _():
        pe_ref[...] = jnp.dot(embe_ref[...], wec_ref[...],
                              preferred_element_type=jnp.float32) + pbc_ref[...]


def _ptables(h, Wd, Ws, embe_pad, wec, pbc):
    blk = lambda i: (i, 0)
    fixed = lambda i: (0, 0)
    return pl.pallas_call(
        _k_ptab,
        grid=(GRID,),
        in_specs=[pl.BlockSpec((NB, HID), blk), pl.BlockSpec((HID, HID), fixed),
                  pl.BlockSpec((HID, HID), fixed), pl.BlockSpec((8, 16), fixed),
                  pl.BlockSpec((16, HID), fixed), pl.BlockSpec((1, HID), fixed)],
        out_specs=[pl.BlockSpec((NB, HID), blk), pl.BlockSpec((NB, HID), blk),
                   pl.BlockSpec((8, HID), fixed)],
        out_shape=[jax.ShapeDtypeStruct((N, HID), jnp.float32),
                   jax.ShapeDtypeStruct((N, HID), jnp.float32),
                   jax.ShapeDtypeStruct((8, HID), jnp.float32)],
    )(h, Wd, Ws, embe_pad, wec, pbc)


_SC_MESH = plsc.VectorSubcoreMesh(core_axis_name="c", subcore_axis_name="s")


@functools.partial(
    pl.kernel, mesh=_SC_MESH,
    compiler_params=pltpu.CompilerParams(use_tc_tiling_on_sc=False),
    out_type=[jax.ShapeDtypeStruct((NPAD * HID,), jnp.float32)] * 4,
    scratch_types=[
        pltpu.VMEM((NSUB * HID,), jnp.float32),
        pltpu.VMEM((NSUB * HID,), jnp.float32),
        pltpu.VMEM((NSUB * HID,), jnp.float32),
        pltpu.VMEM((NSUB * HID,), jnp.float32),
        pltpu.VMEM((NSUB, HID), jnp.float32),
        pltpu.VMEM((2 * CE + 16,), jnp.int32),
        pltpu.VMEM((2 * CE + 16,), jnp.int32),
        pltpu.VMEM((CE,), jnp.int32),
        pltpu.VMEM((CE,), jnp.int32),
        pltpu.VMEM((CE, HID), jnp.float32),
        pltpu.VMEM((CE, HID), jnp.float32),
        pltpu.VMEM((208,), jnp.int32),
        pltpu.VMEM((8, HID), jnp.float32),
        pltpu.SemaphoreType.DMA,
        pltpu.SemaphoreType.DMA,
    ])
def _sc_edge(pd_hbm, ps_hbm, pe_hbm, da_hbm, src_hbm, est_hbm,
             s_hbm, s2_hbm, mx_hbm, mn_hbm,
             sacc, s2acc, mxacc, mnacc, pdloc, da0, da1, sv0, sv1,
             sr0, sr1, estv, pev, sem0, sem1):
    wid = lax.axis_index("s") * 2 + lax.axis_index("c")
    pltpu.sync_copy(est_hbm, estv)
    pltpu.sync_copy(pe_hbm, pev)
    zero16 = jnp.zeros((16,), jnp.float32)
    neg16 = jnp.full((16,), -FMAX, jnp.float32)
    pos16 = jnp.full((16,), FMAX, jnp.float32)

    def stage(base, dab, svb, srb, sem):
        pltpu.sync_copy(da_hbm.at[pl.ds(2 * base, 2 * CE)], dab.at[pl.ds(0, 2 * CE)])
        pltpu.sync_copy(src_hbm.at[pl.ds(base, CE)], svb)
        return pltpu.async_copy(ps_hbm.at[svb], srb, sem)

    for r in range(6):
        j = wid * 6 + r
        ev = estv[pl.ds(j, 16)]
        elo = ev[0]
        ehi = ev[1]

        def initbody(i, _):
            o = pl.ds(i * 16, 16)
            sacc[o] = zero16
            s2acc[o] = zero16
            mxacc[o] = neg16
            mnacc[o] = pos16
            return 0

        lax.fori_loop(0, NSUB * 4, initbody, 0)
        nlo = j * NSUB
        pltpu.sync_copy(pd_hbm.at[pl.ds(nlo, NSUB)], pdloc)
        base0 = (elo // CE) * CE
        nch = (ehi - base0 + CE - 1) // CE

        @pl.when(nch > 0)
        def _():
            stage(base0, da0, sv0, sr0, sem0)

        def process(ci, dab, srb, sem_other, dan, svn, srn):
            base = base0 + ci * CE

            @pl.when(ci + 1 < nch)
            def _():
                stage(base + CE, dan, svn, srn, sem_other)

            lo_i = jnp.maximum(elo - base, 0)
            hi_i = jnp.minimum(ehi - base, CE)

            def edge(c, _):
                v = dab[pl.ds(2 * c, 16)]
                d = v[0]
                a = v[1]
                dloc = d - nlo
                for k in range(4):
                    mk = (pdloc[dloc, pl.ds(16 * k, 16)]
                          + srb[c, pl.ds(16 * k, 16)]
                          + pev[a, pl.ds(16 * k, 16)])
                    o = pl.ds(dloc * HID + 16 * k, 16)
                    sacc[o] += mk
                    s2acc[o] += mk * mk
                    mxacc[o] = jnp.maximum(mxacc[o], mk)
                    mnacc[o] = jnp.minimum(mnacc[o], mk)
                return 0

            lax.fori_loop(lo_i, hi_i, edge, 0)

        def chunk(ci, _):
            @pl.when(ci % 2 == 0)
            def _():
                pltpu.make_async_copy(ps_hbm.at[sv0], sr0, sem0).wait()
                process(ci, da0, sr0, sem1, da1, sv1, sr1)

            @pl.when(ci % 2 == 1)
            def _():
                pltpu.make_async_copy(ps_hbm.at[sv1], sr1, sem1).wait()
                process(ci, da1, sr1, sem0, da0, sv0, sr0)

            return 0

        lax.fori_loop(0, nch, chunk, 0)
        oo = pl.ds(j * NSUB * HID, NSUB * HID)
        pltpu.sync_copy(sacc, s_hbm.at[oo])
        pltpu.sync_copy(s2acc, s2_hbm.at[oo])
        pltpu.sync_copy(mxacc, mx_hbm.at[oo])
        pltpu.sync_copy(mnacc, mn_hbm.at[oo])


def _fuse_weights(lp):
    mix = lp['mix_w']
    parts = []
    for s in range(13):
        bd = block_diag(*[lp['towers'][t]['post_w'][16 * s:16 * s + 16]
                          for t in range(TOW)])
        parts.append(bd @ mix)
    Wp = jnp.concatenate(parts, axis=0)
    pb = jnp.concatenate([lp['towers'][t]['post_b'] for t in range(TOW)])
    bp = (pb @ mix + lp['mix_b']).reshape(1, HID)
    return Wp, bp


def _pre_weights(lp, emb_e):
    Wd = block_diag(*[lp['towers'][t]['pre_w'][0:16] for t in range(TOW)])
    Ws = block_diag(*[lp['towers'][t]['pre_w'][16:32] for t in range(TOW)])
    wec = jnp.concatenate([lp['towers'][t]['pre_w'][32:48] for t in range(TOW)],
                          axis=1)
    pbc = jnp.concatenate([lp['towers'][t]['pre_b'] for t in range(TOW)]).reshape(1, HID)
    embe_pad = jnp.zeros((8, 16), jnp.float32).at[:emb_e.shape[0]].set(emb_e)
    return Wd, Ws, embe_pad, wec, pbc


def kernel(x, batch, edge_index, snorm_n, edge_attr, emb_h, emb_e,
           layer_params, readout_params):
    h = _embed(x, emb_h)
    src = edge_index[0].astype(jnp.int32)
    dst = edge_index[1].astype(jnp.int32)
    perm = jnp.argsort(dst)
    dst_s = jnp.take(dst, perm)
    src_s = jnp.take(src, perm)
    attr_s = jnp.take(edge_attr.astype(jnp.int32), perm)
    row_starts = jnp.searchsorted(dst_s, jnp.arange(N + 1, dtype=jnp.int32)
                                  ).astype(jnp.int32)
    deg2 = (row_starts[1:] - row_starts[:-1]).astype(jnp.float32).reshape(N, 1)
    cuts = jnp.minimum(jnp.arange(NCUT + 1, dtype=jnp.int32) * NSUB, N)
    est = jnp.take(row_starts, cuts)
    est_p = jnp.full((208,), E, jnp.int32).at[:NCUT + 1].set(est)
    padi = jnp.zeros((CE,), jnp.int32)
    src_p = jnp.concatenate([src_s, padi])
    da_p = jnp.stack([jnp.concatenate([dst_s, padi]),
                      jnp.concatenate([attr_s, padi])], axis=1).reshape(-1)
    sn = snorm_n.astype(jnp.float32)
    for lp in layer_params:
        Wd, Ws, embe_pad, wec, pbc = _pre_weights(lp, emb_e)
        pd, ps, pe = _ptables(h, Wd, Ws, embe_pad, wec, pbc)
        pd_pad = jnp.zeros((NPAD, HID), jnp.float32).at[:N].set(pd)
        Sf, S2f, MXf, MNf = _sc_edge(pd_pad, ps, pe, da_p, src_p, est_p)
        S = Sf[:N * HID].reshape(N, HID)
        S2 = S2f[:N * HID].reshape(N, HID)
        MX = MXf[:N * HID].reshape(N, HID)
        MN = MNf[:N * HID].reshape(N, HID)
        Wp, bp = _fuse_weights(lp)
        hm, bn = _node_stage(h, S, S2, MX, MN, deg2, sn, Wp, bp)
        h = _bn_stage(hm, h, bn, lp['bn_g'], lp['bn_b'])
    return _readout(h, batch, readout_params)
